# baseline (speedup 1.0000x reference)
import numpy as np
import jax
import jax.numpy as jnp
from jax.experimental import pallas as pl
from jax.experimental.pallas import tpu as pltpu

_B = 2048
_IN = 128
_HID = 512
_LAT = 32
_NC = 8
_KC = 64

_STREAMED = (
    ((_HID, _LAT), jnp.float32),
    ((1, _LAT), jnp.float32),
    ((_NC, _KC, _LAT), jnp.float32),
    ((_LAT, _LAT // 2), jnp.float32),
    ((1, _LAT // 2), jnp.float32),
    ((_LAT // 2, _LAT), jnp.float32),
    ((1, _LAT), jnp.float32),
    ((_LAT, _NC), jnp.float32),
    ((1, _NC), jnp.float32),
    ((_NC, _HID, _LAT), jnp.float32),
    ((_NC, _HID), jnp.float32),
    ((_HID, _HID), jnp.float32),
    ((1, _HID), jnp.float32),
    ((_HID, _IN), jnp.float32),
    ((1, _IN), jnp.float32),
    ((_HID, _IN), jnp.float32),
    ((1, _IN), jnp.float32),
    ((_LAT, _IN), jnp.float32),
    ((1, _IN), jnp.float32),
    ((1, 1), jnp.float32),
)
_NS = len(_STREAMED)


def _gelu(x):
    return x * 0.5 * (1.0 + jax.lax.erf(x * np.float32(1.0 / np.sqrt(2.0))))


def _softmax(x):
    m = jnp.max(x, axis=1, keepdims=True)
    e = jnp.exp(x - m)
    return e / jnp.sum(e, axis=1, keepdims=True)


def _iota2(shape, dim):
    return jax.lax.broadcasted_iota(jnp.int32, shape, dim)


def _blockdiag(tile, n):
    r, c = tile.shape
    row = jnp.concatenate([tile] * n, axis=1)
    full = jnp.concatenate([row] * n, axis=0)
    shape = (n * r, n * c)
    mask = (_iota2(shape, 0) // r) == (_iota2(shape, 1) // c)
    return jnp.where(mask, full, 0.0)


def _fwd(x_ref, W1, b1, W2, b2, Wk, bk, cq, *rest):
    hbm = rest[:_NS]
    xhat_ref, vq_ref, enc_ref, dec_ref, kc_ref = rest[_NS:_NS + 5]
    vmem = rest[_NS + 5:2 * _NS + 5]
    sems = rest[2 * _NS + 5]
    f32 = jnp.float32

    copies = [pltpu.make_async_copy(hbm[i], vmem[i], sems.at[i])
              for i in range(_NS)]
    for c in copies:
        c.start()

    def dot(a, b):
        return jnp.dot(a, b, preferred_element_type=f32)

    def dot_t(a, b):
        return jax.lax.dot_general(a, b, (((1,), (1,)), ((), ())),
                                   preferred_element_type=f32)

    def bdot(a, b):
        return jnp.dot(a.astype(jnp.bfloat16), b.astype(jnp.bfloat16),
                       preferred_element_type=f32)

    cqt = cq[...].T
    x = x_ref[...]
    bb = x.shape[0]
    f = _gelu(dot(x, W1[...]) + b1[...])
    f = _gelu(dot(f, W2[...]) + b2[...])
    k = dot(f, Wk[...]) + bk[...]
    scores = dot(k, cqt) / f32(np.sqrt(_HID))
    enc_rw = _softmax(scores)
    kc_ref[...] = jnp.argmax(enc_rw, axis=1).astype(jnp.int32)[:, None]

    for c in copies:
        c.wait()
    (Wv, bv, cb3, Ws1, bs1, Ws2, bs2, Wr, br, cw3, cbias,
     Wr1, br1, Wr2, br2, Wskip, bskip, Wt, bt, ts) = vmem

    cb = cb3[...].reshape(_NC * _KC, _LAT)
    cbn = jnp.sum(cb * cb, axis=1, keepdims=True)
    cb_aug = jnp.concatenate([cb, cbn], axis=1)
    cb_bd = jnp.where(
        (_iota2((_NC * _KC, _NC * _LAT), 0) // _KC)
        == (_iota2((_NC * _KC, _NC * _LAT), 1) // _LAT),
        jnp.concatenate([cb] * _NC, axis=1), 0.0)
    Ws1bd = _blockdiag(Ws1[...], _NC)
    Ws2bd = _blockdiag(Ws2[...], _NC)
    bs1t = jnp.concatenate([bs1[...]] * _NC, axis=1)
    bs2t = jnp.concatenate([bs2[...]] * _NC, axis=1)
    E = ((_iota2((_NC, _NC * _LAT), 1) // _LAT)
         == _iota2((_NC, _NC * _LAT), 0)).astype(f32)
    S = ((_iota2((_NC * _LAT, _LAT), 0) % _LAT)
         == _iota2((_NC * _LAT, _LAT), 1)).astype(f32)
    cw = cw3[...]
    CW = jnp.concatenate([cw[c].T for c in range(_NC)], axis=0)

    v = dot(f, Wv[...]) + bv[...]

    v_aug = jnp.concatenate([v * f32(-2.0), jnp.ones((bb, 1), f32)], axis=1)
    dist = dot_t(v_aug, cb_aug)
    ohs = []
    for c in range(_NC):
        sl = dist[:, c * _KC:(c + 1) * _KC]
        ohs.append((sl == jnp.min(sl, axis=1, keepdims=True)).astype(f32))
    OH = jnp.concatenate(ohs, axis=1)
    ZQ = bdot(OH, cb_bd)

    w_exp = dot(enc_rw, E)
    D = jnp.concatenate([v] * _NC, axis=1) - ZQ
    vq_ref[...] = (jnp.sum(D * D * w_exp, keepdims=True)
                   * f32(1.25 / (_B * _LAT)))

    h = _gelu(bdot(D, Ws1bd) + bs1t)
    ZN = bdot(h, Ws2bd) + bs2t
    z_geo = dot((ZQ + ZN) * w_exp, S)
    z_tex = v - z_geo
    zg = jnp.tanh(z_geo)
    logits = dot(zg, Wr[...]) + br[...]
    dec_rw = _softmax(logits)
    wd = dot(dec_rw, E)
    ZGW = jnp.concatenate([zg] * _NC, axis=1) * wd
    hg = bdot(ZGW, CW) + dot(dec_rw, cbias[...])
    r = _gelu(hg)
    r = _gelu(bdot(r, Wr1[...]) + br1[...])
    tsc = ts[0, 0]
    xhat_ref[...] = (bdot(r, Wr2[...]) + bdot(hg, Wskip[...])
                     + bdot(jnp.tanh(z_tex) * tsc, Wt[...])
                     + (br2[...] + bskip[...] + tsc * bt[...]))
    enc_ref[...] = enc_rw
    dec_ref[...] = dec_rw


def kernel(x, params):
    p = params
    args = (
        x,
        p['W1'], p['b1'][None], p['W2'], p['b2'][None],
        p['Wk'], p['bk'][None], p['chart_queries'],
        p['Wv'], p['bv'][None], p['codebook'],
        p['Ws1'], p['bs1'][None], p['Ws2'], p['bs2'][None],
        p['Wr'], p['br'][None], p['chart_weight'], p['chart_bias'],
        p['Wr1'], p['br1'][None], p['Wr2'], p['br2'][None],
        p['Wskip'], p['bskip'][None], p['Wt'], p['bt'][None],
        jnp.reshape(p['tex_scale'], (1, 1)),
    )

    def full(a):
        nd = a.ndim
        return pl.BlockSpec(a.shape, lambda i, _n=nd: (0,) * _n)

    in_specs = [pl.BlockSpec((_B, _IN), lambda i: (i, 0))]
    in_specs += [full(a) for a in args[1:8]]
    in_specs += [pl.BlockSpec(memory_space=pl.ANY)] * _NS
    out_specs = [
        pl.BlockSpec((_B, _IN), lambda i: (i, 0)),
        pl.BlockSpec((1, 1), lambda i: (0, 0)),
        pl.BlockSpec((_B, _NC), lambda i: (i, 0)),
        pl.BlockSpec((_B, _NC), lambda i: (i, 0)),
        pl.BlockSpec((_B, 1), lambda i: (i, 0)),
    ]
    out_shape = [
        jax.ShapeDtypeStruct((_B, _IN), jnp.float32),
        jax.ShapeDtypeStruct((1, 1), jnp.float32),
        jax.ShapeDtypeStruct((_B, _NC), jnp.float32),
        jax.ShapeDtypeStruct((_B, _NC), jnp.float32),
        jax.ShapeDtypeStruct((_B, 1), jnp.int32),
    ]
    scratch_shapes = [pltpu.VMEM(s, d) for s, d in _STREAMED]
    scratch_shapes.append(pltpu.SemaphoreType.DMA((_NS,)))
    xh, vq, enc, dec, kc = pl.pallas_call(
        _fwd,
        grid=(1,),
        in_specs=in_specs,
        out_specs=out_specs,
        out_shape=out_shape,
        scratch_shapes=scratch_shapes,
    )(*args)
    return xh, vq[0, 0], enc, dec, kc[:, 0]


# speedup vs baseline: 1.0003x; 1.0003x over previous
import numpy as np
import jax
import jax.numpy as jnp
from jax.experimental import pallas as pl
from jax.experimental.pallas import tpu as pltpu

_B = 2048
_IN = 128
_HID = 512
_LAT = 32
_NC = 8
_KC = 64

_STREAMED = (
    ((_NC, _HID, _LAT), jnp.float32),
    ((_HID, _HID), jnp.float32),
    ((_HID, _IN), jnp.float32),
    ((_HID, _IN), jnp.float32),
)
_NS = len(_STREAMED)


def _gelu(x):
    return x * 0.5 * (1.0 + jax.lax.erf(x * np.float32(1.0 / np.sqrt(2.0))))


def _softmax(x):
    m = jnp.max(x, axis=1, keepdims=True)
    e = jnp.exp(x - m)
    return e / jnp.sum(e, axis=1, keepdims=True)


def _iota2(shape, dim):
    return jax.lax.broadcasted_iota(jnp.int32, shape, dim)


def _blockdiag(tile, n):
    r, c = tile.shape
    row = jnp.concatenate([tile] * n, axis=1)
    full = jnp.concatenate([row] * n, axis=0)
    shape = (n * r, n * c)
    mask = (_iota2(shape, 0) // r) == (_iota2(shape, 1) // c)
    return jnp.where(mask, full, 0.0)


def _fwd(x_ref, W1, b1, W2, b2, Wk, bk, cq,
         Wv, bv, cb3, Ws1, bs1, Ws2, bs2, Wr, br, cbias,
         br1, br2, bskip, Wt, bt, ts, *rest):
    hbm = rest[:_NS]
    xhat_ref, vq_ref, enc_ref, dec_ref, kc_ref = rest[_NS:_NS + 5]
    vmem = rest[_NS + 5:2 * _NS + 5]
    sems = rest[2 * _NS + 5]
    f32 = jnp.float32

    copies = [pltpu.make_async_copy(hbm[i], vmem[i], sems.at[i])
              for i in range(_NS)]
    for c in copies:
        c.start()

    def dot(a, b):
        return jnp.dot(a, b, preferred_element_type=f32)

    def dot_t(a, b):
        return jax.lax.dot_general(a, b, (((1,), (1,)), ((), ())),
                                   preferred_element_type=f32)

    def bdot(a, b):
        return jnp.dot(a.astype(jnp.bfloat16), b.astype(jnp.bfloat16),
                       preferred_element_type=f32)

    cqt = cq[...].T
    x = x_ref[...]
    bb = x.shape[0]
    f = _gelu(dot(x, W1[...]) + b1[...])
    f = _gelu(dot(f, W2[...]) + b2[...])
    k = dot(f, Wk[...]) + bk[...]
    scores = dot(k, cqt) / f32(np.sqrt(_HID))
    enc_rw = _softmax(scores)
    kc_ref[...] = jnp.argmax(enc_rw, axis=1).astype(jnp.int32)[:, None]

    for c in copies:
        c.wait()
    cw3, Wr1, Wr2, Wskip = vmem

    cb = cb3[...].reshape(_NC * _KC, _LAT)
    cbn = jnp.sum(cb * cb, axis=1, keepdims=True)
    cb_aug = jnp.concatenate([cb, cbn], axis=1)
    cb_bd = jnp.where(
        (_iota2((_NC * _KC, _NC * _LAT), 0) // _KC)
        == (_iota2((_NC * _KC, _NC * _LAT), 1) // _LAT),
        jnp.concatenate([cb] * _NC, axis=1), 0.0)
    Ws1bd = _blockdiag(Ws1[...], _NC)
    Ws2bd = _blockdiag(Ws2[...], _NC)
    bs1t = jnp.concatenate([bs1[...]] * _NC, axis=1)
    bs2t = jnp.concatenate([bs2[...]] * _NC, axis=1)
    E = ((_iota2((_NC, _NC * _LAT), 1) // _LAT)
         == _iota2((_NC, _NC * _LAT), 0)).astype(f32)
    S = ((_iota2((_NC * _LAT, _LAT), 0) % _LAT)
         == _iota2((_NC * _LAT, _LAT), 1)).astype(f32)
    cw = cw3[...]
    CW = jnp.concatenate([cw[c].T for c in range(_NC)], axis=0)

    v = dot(f, Wv[...]) + bv[...]

    v_aug = jnp.concatenate([v * f32(-2.0), jnp.ones((bb, 1), f32)], axis=1)
    dist = dot_t(v_aug, cb_aug)
    ohs = []
    for c in range(_NC):
        sl = dist[:, c * _KC:(c + 1) * _KC]
        ohs.append((sl == jnp.min(sl, axis=1, keepdims=True)).astype(f32))
    OH = jnp.concatenate(ohs, axis=1)
    ZQ = bdot(OH, cb_bd)

    w_exp = dot(enc_rw, E)
    D = jnp.concatenate([v] * _NC, axis=1) - ZQ
    vq_ref[...] = (jnp.sum(D * D * w_exp, keepdims=True)
                   * f32(1.25 / (_B * _LAT)))

    h = _gelu(bdot(D, Ws1bd) + bs1t)
    ZN = bdot(h, Ws2bd) + bs2t
    z_geo = dot((ZQ + ZN) * w_exp, S)
    z_tex = v - z_geo
    zg = jnp.tanh(z_geo)
    logits = dot(zg, Wr[...]) + br[...]
    dec_rw = _softmax(logits)
    wd = dot(dec_rw, E)
    ZGW = jnp.concatenate([zg] * _NC, axis=1) * wd
    hg = bdot(ZGW, CW) + dot(dec_rw, cbias[...])
    r = _gelu(hg)
    r = _gelu(bdot(r, Wr1[...]) + br1[...])
    tsc = ts[0, 0]
    xhat_ref[...] = (bdot(r, Wr2[...]) + bdot(hg, Wskip[...])
                     + bdot(jnp.tanh(z_tex) * tsc, Wt[...])
                     + (br2[...] + bskip[...] + tsc * bt[...]))
    enc_ref[...] = enc_rw
    dec_ref[...] = dec_rw


def kernel(x, params):
    p = params
    args = (
        x,
        p['W1'], p['b1'][None], p['W2'], p['b2'][None],
        p['Wk'], p['bk'][None], p['chart_queries'],
        p['Wv'], p['bv'][None], p['codebook'],
        p['Ws1'], p['bs1'][None], p['Ws2'], p['bs2'][None],
        p['Wr'], p['br'][None], p['chart_bias'],
        p['br1'][None], p['br2'][None], p['bskip'][None],
        p['Wt'], p['bt'][None],
        jnp.reshape(p['tex_scale'], (1, 1)),
        p['chart_weight'], p['Wr1'], p['Wr2'], p['Wskip'],
    )

    def full(a):
        nd = a.ndim
        return pl.BlockSpec(a.shape, lambda i, _n=nd: (0,) * _n)

    in_specs = [pl.BlockSpec((_B, _IN), lambda i: (i, 0))]
    in_specs += [full(a) for a in args[1:len(args) - _NS]]
    in_specs += [pl.BlockSpec(memory_space=pl.ANY)] * _NS
    out_specs = [
        pl.BlockSpec((_B, _IN), lambda i: (i, 0)),
        pl.BlockSpec((1, 1), lambda i: (0, 0)),
        pl.BlockSpec((_B, _NC), lambda i: (i, 0)),
        pl.BlockSpec((_B, _NC), lambda i: (i, 0)),
        pl.BlockSpec((_B, 1), lambda i: (i, 0)),
    ]
    out_shape = [
        jax.ShapeDtypeStruct((_B, _IN), jnp.float32),
        jax.ShapeDtypeStruct((1, 1), jnp.float32),
        jax.ShapeDtypeStruct((_B, _NC), jnp.float32),
        jax.ShapeDtypeStruct((_B, _NC), jnp.float32),
        jax.ShapeDtypeStruct((_B, 1), jnp.int32),
    ]
    scratch_shapes = [pltpu.VMEM(s, d) for s, d in _STREAMED]
    scratch_shapes.append(pltpu.SemaphoreType.DMA((_NS,)))
    xh, vq, enc, dec, kc = pl.pallas_call(
        _fwd,
        grid=(1,),
        in_specs=in_specs,
        out_specs=out_specs,
        out_shape=out_shape,
        scratch_shapes=scratch_shapes,
    )(*args)
    return xh, vq[0, 0], enc, dec, kc[:, 0]
